# baseline (speedup 1.0000x reference)
import functools

import jax
import jax.numpy as jnp
from jax import lax
from jax.experimental import pallas as pl
from jax.experimental.pallas import tpu as pltpu
from jax.experimental.pallas import tpu_sc as plsc

N = 10000
E = 160000
D = 256
H = 16
W128 = 128

NC = 2
NS = 16
NW = NC * NS
R = 320
NPAD = NW * R
G = 256
KCH = 1600
CAPB = 3072
CSEL = 161280
NEG = float("-inf")

_mesh = plsc.VectorSubcoreMesh(core_axis_name="c", subcore_axis_name="s")
_params = pltpu.CompilerParams(needs_layout_passes=False, use_tc_tiling_on_sc=False)


def _wid():
    return lax.axis_index("s") * NC + lax.axis_index("c")


def _pre_body(src_hbm, dst_hbm, opair, ocnt, srcb, dstb, bs, bd, cntv,
              psem0, psem1):
    wid = _wid()
    lo = wid * R
    lane = jnp.arange(16, dtype=jnp.int32)
    sent_d = jnp.full((16,), R, jnp.int32)
    psems = (psem0, psem1)
    NCHUNK = E // KCH

    def load(c, slot):
        pltpu.async_copy(src_hbm.at[pl.ds(pl.multiple_of(c * KCH, 8), KCH)],
                         srcb.at[pl.ds(slot * KCH, KCH)], psems[slot])
        pltpu.async_copy(dst_hbm.at[pl.ds(pl.multiple_of(c * KCH, 8), KCH)],
                         dstb.at[pl.ds(slot * KCH, KCH)], psems[slot])

    def wait_load(c, slot):
        pltpu.make_async_copy(src_hbm.at[pl.ds(pl.multiple_of(c * KCH, 8), KCH)],
                              srcb.at[pl.ds(slot * KCH, KCH)], psems[slot]).wait()
        pltpu.make_async_copy(dst_hbm.at[pl.ds(pl.multiple_of(c * KCH, 8), KCH)],
                              dstb.at[pl.ds(slot * KCH, KCH)], psems[slot]).wait()

    def filter_chunk(slot, nb, off):
        def vec_body(i, nb):
            vs = []
            for u in range(4):
                s = pl.ds(pl.multiple_of(slot * KCH + (i * 4 + u) * 16, 16), 16)
                sv = srcb[s]
                dv = dstb[s]
                m = (dv >= lo) & (dv < lo + R)
                cum = plsc.cumsum(m.astype(jnp.int32))
                vs.append((sv, dv, m, cum))
            b = nb
            for sv, dv, m, cum in vs:
                pos = b + cum - 1
                plsc.store_scatter(bs, [pos], sv, mask=m)
                plsc.store_scatter(bd, [pos], dv - lo, mask=m)
                b = b + cum[15]
            return b

        nb = lax.fori_loop(0, KCH // 64, vec_body, nb)

        nflush = nb // G

        def fl(j, _):
            pltpu.sync_copy(bs.at[pl.ds(pl.multiple_of(j * G, 256), G)],
                            opair.at[pl.ds(pl.multiple_of(2 * (wid * CSEL + off + j * G), 256), G)])
            pltpu.sync_copy(bd.at[pl.ds(pl.multiple_of(j * G, 256), G)],
                            opair.at[pl.ds(pl.multiple_of(2 * (wid * CSEL + off + j * G) + G, 256), G)])
            return 0

        lax.fori_loop(0, nflush, fl, 0)
        rem = nb - nflush * G

        def mv(i, _):
            bs[pl.ds(pl.multiple_of(i * 16, 16), 16)] = bs[pl.ds(pl.multiple_of(nflush * G + i * 16, 16), 16)]
            bd[pl.ds(pl.multiple_of(i * 16, 16), 16)] = bd[pl.ds(pl.multiple_of(nflush * G + i * 16, 16), 16)]
            return 0

        lax.fori_loop(0, (rem + 15) // 16, mv, 0)
        return rem, off + nflush * G

    load(jnp.int32(0), 0)
    load(jnp.int32(1), 1)

    def cb(c2, carry):
        nb, off = carry
        c0 = c2 * 2
        wait_load(c0, 0)
        nb, off = filter_chunk(0, nb, off)

        @pl.when(c0 + 2 < NCHUNK)
        def _():
            load(c0 + 2, 0)

        wait_load(c0 + 1, 1)
        nb, off = filter_chunk(1, nb, off)

        @pl.when(c0 + 3 < NCHUNK)
        def _():
            load(c0 + 3, 1)

        return nb, off

    nb, off = lax.fori_loop(0, NCHUNK // 2, cb, (jnp.int32(0), jnp.int32(0)))

    base = (nb // 16) * 16
    m = lane < (nb - base)
    base = pl.multiple_of(base, 16)
    bs[pl.ds(base, 16)] = jnp.where(m, bs[pl.ds(base, 16)], lane)
    bd[pl.ds(base, 16)] = jnp.where(m, bd[pl.ds(base, 16)], sent_d)
    nb16 = base + 16
    nb_pad = ((nb16 + G - 1) // G) * G

    def fb(i, _):
        bs[pl.ds(pl.multiple_of(nb16 + i * 16, 16), 16)] = lane
        bd[pl.ds(pl.multiple_of(nb16 + i * 16, 16), 16)] = sent_d
        return 0

    lax.fori_loop(0, (nb_pad - nb16) // 16, fb, 0)

    def fl2(j, _):
        pltpu.sync_copy(bs.at[pl.ds(pl.multiple_of(j * G, 256), G)],
                        opair.at[pl.ds(pl.multiple_of(2 * (wid * CSEL + off + j * G), 256), G)])
        pltpu.sync_copy(bd.at[pl.ds(pl.multiple_of(j * G, 256), G)],
                        opair.at[pl.ds(pl.multiple_of(2 * (wid * CSEL + off + j * G) + G, 256), G)])
        return 0

    lax.fori_loop(0, nb_pad // G, fl2, 0)

    cntv[pl.ds(0, 16)] = jnp.full((16,), off + nb, jnp.int32)
    pltpu.sync_copy(cntv, ocnt.at[pl.ds(pl.multiple_of(wid * 16, 16), 16)])


_preprocess = functools.partial(
    pl.kernel,
    out_type=(
        jax.ShapeDtypeStruct((2 * NW * CSEL,), jnp.int32),
        jax.ShapeDtypeStruct((NW * 16,), jnp.int32),
    ),
    mesh=_mesh,
    compiler_params=_params,
    scratch_types=[
        pltpu.VMEM((2 * KCH,), jnp.int32),
        pltpu.VMEM((2 * KCH,), jnp.int32),
        pltpu.VMEM((CAPB,), jnp.int32),
        pltpu.VMEM((CAPB,), jnp.int32),
        pltpu.VMEM((16,), jnp.int32),
        pltpu.SemaphoreType.DMA,
        pltpu.SemaphoreType.DMA,
    ],
)(_pre_body)


def _agg_body(rw, mult, h2d, opair, ocnt, out, sd, gidx, rows,
              agg, cnt16, sem0, sem1):
    wid = _wid()
    nsub = rw // 16
    pltpu.sync_copy(ocnt.at[pl.ds(pl.multiple_of(wid * 16, 16), 16)], cnt16)
    cnt = cnt16[pl.ds(0, 16)][0]
    ngroups = (cnt + G - 1) // G
    neg16 = jnp.full((16,), NEG, jnp.float32)
    sems = (sem0, sem1)

    for p in range(mult):
        def ib(r, _):
            agg[pl.ds(pl.multiple_of(r * 16, 16), 16)] = neg16
            return 0

        lax.fori_loop(0, (R + 1) * nsub, ib, 0)

        def fetch(g, slot):
            pltpu.sync_copy(opair.at[pl.ds(pl.multiple_of(2 * (wid * CSEL + g * G), 256), 2 * G)],
                            sd.at[pl.ds(slot * 2 * G, 2 * G)])
            if mult > 1:
                def cv(i, _):
                    s = pl.ds(slot * 2 * G + i * 16, 16)
                    gidx[pl.ds(slot * G + i * 16, 16)] = sd[s] * mult + p
                    return 0
                lax.fori_loop(0, G // 16, cv, 0)
                for q in range(G // 128):
                    pltpu.async_copy(h2d.at[gidx.at[pl.ds(slot * G + q * 128, 128)]],
                                     rows.at[pl.ds(slot * G + q * 128, 128)],
                                     sems[slot])
            else:
                for q in range(G // 128):
                    pltpu.async_copy(h2d.at[sd.at[pl.ds(slot * 2 * G + q * 128, 128)]],
                                     rows.at[pl.ds(slot * G + q * 128, 128)],
                                     sems[slot])

        def drain_rmw(slot):
            for q in range(G // 128):
                pltpu.make_async_copy(
                    h2d.at[sd.at[pl.ds(slot * 2 * G + q * 128, 128)]],
                    rows.at[pl.ds(slot * G + q * 128, 128)],
                    sems[slot]).wait()

            def eb(e16, _):
                dv = sd[pl.ds(pl.multiple_of(slot * 2 * G + G + e16 * 16, 16), 16)]
                dbase = dv * rw
                for t in range(16):
                    db = dbase[t]
                    e = slot * G + e16 * 16 + t
                    for j in range(nsub):
                        s = pl.ds(db + j * 16, 16)
                        agg[s] = jnp.maximum(agg[s], rows[e, pl.ds(j * 16, 16)])
                return 0

            lax.fori_loop(0, G // 16, eb, 0)

        @pl.when(ngroups > 0)
        def _():
            fetch(jnp.int32(0), 0)

        def gp(i, _):
            g1 = i * 2 + 1

            @pl.when(g1 < ngroups)
            def _():
                fetch(g1, 1)

            drain_rmw(0)

            @pl.when(g1 < ngroups)
            def _():
                @pl.when(g1 + 1 < ngroups)
                def _():
                    fetch(g1 + 1, 0)

                drain_rmw(1)

            return 0

        lax.fori_loop(0, (ngroups + 1) // 2, gp, 0)

        def wb(r, _):
            s = pl.ds(pl.multiple_of(r * 16, 16), 16)
            v = agg[s]
            agg[s] = jnp.where(v == NEG, jnp.float32(0.0), v)
            return 0

        lax.fori_loop(0, R * nsub, wb, 0)
        pltpu.sync_copy(agg.at[pl.ds(0, R * rw)],
                        out.at[p, pl.ds(pl.multiple_of(wid * R * rw, 256), R * rw)])


def _make_agg(rw, mult):
    return functools.partial(
        pl.kernel,
        out_type=jax.ShapeDtypeStruct((mult, NPAD * rw), jnp.float32),
        mesh=_mesh,
        compiler_params=_params,
        scratch_types=[
            pltpu.VMEM((4 * G,), jnp.int32),
            pltpu.VMEM((2 * G,), jnp.int32),
            pltpu.VMEM((2 * G, rw), jnp.float32),
            pltpu.VMEM(((R + 1) * rw,), jnp.float32),
            pltpu.VMEM((16,), jnp.int32),
            pltpu.SemaphoreType.DMA,
            pltpu.SemaphoreType.DMA,
        ],
    )(functools.partial(_agg_body, rw, mult))


_agg_l1 = _make_agg(128, 2)
_agg_sm = _make_agg(16, 1)


def _dense1_body(a0_ref, a1_ref, x_ref, w0_ref, w1_ref, wr_ref, b_ref, o_ref):
    acc = jnp.dot(a0_ref[...], w0_ref[...], preferred_element_type=jnp.float32)
    acc += jnp.dot(a1_ref[...], w1_ref[...], preferred_element_type=jnp.float32)
    acc += jnp.dot(x_ref[...], wr_ref[...], preferred_element_type=jnp.float32)
    o_ref[...] = jnp.maximum(acc + b_ref[...], 0.0)


def _dense_body(a_ref, h_ref, wl_ref, wr_ref, b_ref, o_ref):
    acc = jnp.dot(a_ref[...], wl_ref[...], preferred_element_type=jnp.float32)
    acc += jnp.dot(h_ref[...], wr_ref[...], preferred_element_type=jnp.float32)
    o_ref[...] = jnp.maximum(acc + b_ref[...], 0.0)


def _last_body(a_ref, h_ref, wl_ref, wr_ref, b_ref, o_ref):
    acc = jnp.dot(a_ref[...], wl_ref[...], preferred_element_type=jnp.float32)
    acc += jnp.dot(h_ref[...], wr_ref[...], preferred_element_type=jnp.float32)
    z = acc + b_ref[...]
    m = jnp.max(z, axis=1, keepdims=True)
    ez = jnp.exp(z - m)
    o_ref[...] = z - m - jnp.log(jnp.sum(ez, axis=1, keepdims=True))


def _dense1(a0, a1, xp, w0, w1, wr, b):
    return pl.pallas_call(
        _dense1_body,
        out_shape=jax.ShapeDtypeStruct((NPAD, H), jnp.float32),
    )(a0, a1, xp, w0, w1, wr, b)


def _dense(a, h, wl, wr, b):
    return pl.pallas_call(
        _dense_body,
        out_shape=jax.ShapeDtypeStruct((NPAD, H), jnp.float32),
    )(a, h, wl, wr, b)


def _last(a, h, wl, wr, b):
    return pl.pallas_call(
        _last_body,
        out_shape=jax.ShapeDtypeStruct((NPAD, H), jnp.float32),
    )(a, h, wl, wr, b)


def kernel(x, edge_index, Wl1, bl1, Wr1, Wl2, bl2, Wr2, Wl3, bl3, Wr3,
           Wl4, bl4, Wr4, Wl5, bl5, Wr5, Wl6, bl6, Wr6, Wl7, bl7, Wr7):
    src = edge_index[0]
    dst = edge_index[1]
    opair, ocnt = _preprocess(src, dst)

    xp = jnp.pad(x, ((0, NPAD - N), (0, 0)))
    x2 = xp.reshape(NPAD * 2, 128)

    aggs = _agg_l1(x2, opair, ocnt).reshape(2, NPAD, 128)
    h = _dense1(aggs[0], aggs[1], xp,
                Wl1.T[:128, :], Wl1.T[128:, :], Wr1.T, bl1.reshape(1, H))

    layers = [(Wl2, bl2, Wr2), (Wl3, bl3, Wr3), (Wl4, bl4, Wr4),
              (Wl5, bl5, Wr5), (Wl6, bl6, Wr6)]
    for Wl, bl, Wr in layers:
        agg = _agg_sm(h, opair, ocnt)[0].reshape(NPAD, H)
        h = _dense(agg, h, Wl.T, Wr.T, bl.reshape(1, H))

    agg = _agg_sm(h, opair, ocnt)[0].reshape(NPAD, H)
    out = _last(agg, h, Wl7.T, Wr7.T, bl7.reshape(1, H))
    return out[:N]


# speedup vs baseline: 1.0950x; 1.0950x over previous
import functools

import jax
import jax.numpy as jnp
from jax import lax
from jax.experimental import pallas as pl
from jax.experimental.pallas import tpu as pltpu
from jax.experimental.pallas import tpu_sc as plsc

N = 10000
E = 160000
D = 256
H = 16
W128 = 128

NC = 2
NS = 16
NW = NC * NS
R = 320
NPAD = NW * R
G = 256
KCH = 1600
CAPB = 3072
CSEL = 161280
NEG = float("-inf")

_mesh = plsc.VectorSubcoreMesh(core_axis_name="c", subcore_axis_name="s")
_params = pltpu.CompilerParams(needs_layout_passes=False, use_tc_tiling_on_sc=False)


def _wid():
    return lax.axis_index("s") * NC + lax.axis_index("c")


def _pre_body(src_hbm, dst_hbm, opair, ocnt, srcb, dstb, bs, bd, cntv,
              psem0, psem1):
    wid = _wid()
    lo = wid * R
    lane = jnp.arange(16, dtype=jnp.int32)
    sent_d = jnp.full((16,), R, jnp.int32)
    psems = (psem0, psem1)
    NCHUNK = E // KCH

    def load(c, slot):
        pltpu.async_copy(src_hbm.at[pl.ds(pl.multiple_of(c * KCH, 8), KCH)],
                         srcb.at[pl.ds(slot * KCH, KCH)], psems[slot])
        pltpu.async_copy(dst_hbm.at[pl.ds(pl.multiple_of(c * KCH, 8), KCH)],
                         dstb.at[pl.ds(slot * KCH, KCH)], psems[slot])

    def wait_load(c, slot):
        pltpu.make_async_copy(src_hbm.at[pl.ds(pl.multiple_of(c * KCH, 8), KCH)],
                              srcb.at[pl.ds(slot * KCH, KCH)], psems[slot]).wait()
        pltpu.make_async_copy(dst_hbm.at[pl.ds(pl.multiple_of(c * KCH, 8), KCH)],
                              dstb.at[pl.ds(slot * KCH, KCH)], psems[slot]).wait()

    def filter_chunk(slot, nb, off):
        def vec_body(i, nb):
            vs = []
            for u in range(4):
                s = pl.ds(pl.multiple_of(slot * KCH + (i * 4 + u) * 16, 16), 16)
                sv = srcb[s]
                dv = dstb[s]
                m = (dv >= lo) & (dv < lo + R)
                cum = plsc.cumsum(m.astype(jnp.int32))
                vs.append((sv, dv, m, cum))
            b = nb
            for sv, dv, m, cum in vs:
                pos = b + cum - 1
                plsc.store_scatter(bs, [pos], sv, mask=m)
                plsc.store_scatter(bd, [pos], dv - lo, mask=m)
                b = b + cum[15]
            return b

        nb = lax.fori_loop(0, KCH // 64, vec_body, nb)

        nflush = nb // G

        def fl(j, _):
            pltpu.sync_copy(bs.at[pl.ds(pl.multiple_of(j * G, 256), G)],
                            opair.at[pl.ds(pl.multiple_of(2 * (wid * CSEL + off + j * G), 256), G)])
            pltpu.sync_copy(bd.at[pl.ds(pl.multiple_of(j * G, 256), G)],
                            opair.at[pl.ds(pl.multiple_of(2 * (wid * CSEL + off + j * G) + G, 256), G)])
            return 0

        lax.fori_loop(0, nflush, fl, 0)
        rem = nb - nflush * G

        def mv(i, _):
            bs[pl.ds(pl.multiple_of(i * 16, 16), 16)] = bs[pl.ds(pl.multiple_of(nflush * G + i * 16, 16), 16)]
            bd[pl.ds(pl.multiple_of(i * 16, 16), 16)] = bd[pl.ds(pl.multiple_of(nflush * G + i * 16, 16), 16)]
            return 0

        lax.fori_loop(0, (rem + 15) // 16, mv, 0)
        return rem, off + nflush * G

    load(jnp.int32(0), 0)
    load(jnp.int32(1), 1)

    def cb(c2, carry):
        nb, off = carry
        c0 = c2 * 2
        wait_load(c0, 0)
        nb, off = filter_chunk(0, nb, off)

        @pl.when(c0 + 2 < NCHUNK)
        def _():
            load(c0 + 2, 0)

        wait_load(c0 + 1, 1)
        nb, off = filter_chunk(1, nb, off)

        @pl.when(c0 + 3 < NCHUNK)
        def _():
            load(c0 + 3, 1)

        return nb, off

    nb, off = lax.fori_loop(0, NCHUNK // 2, cb, (jnp.int32(0), jnp.int32(0)))

    base = (nb // 16) * 16
    m = lane < (nb - base)
    base = pl.multiple_of(base, 16)
    bs[pl.ds(base, 16)] = jnp.where(m, bs[pl.ds(base, 16)], lane)
    bd[pl.ds(base, 16)] = jnp.where(m, bd[pl.ds(base, 16)], sent_d)
    nb16 = base + 16
    nb_pad = ((nb16 + G - 1) // G) * G

    def fb(i, _):
        bs[pl.ds(pl.multiple_of(nb16 + i * 16, 16), 16)] = lane
        bd[pl.ds(pl.multiple_of(nb16 + i * 16, 16), 16)] = sent_d
        return 0

    lax.fori_loop(0, (nb_pad - nb16) // 16, fb, 0)

    def fl2(j, _):
        pltpu.sync_copy(bs.at[pl.ds(pl.multiple_of(j * G, 256), G)],
                        opair.at[pl.ds(pl.multiple_of(2 * (wid * CSEL + off + j * G), 256), G)])
        pltpu.sync_copy(bd.at[pl.ds(pl.multiple_of(j * G, 256), G)],
                        opair.at[pl.ds(pl.multiple_of(2 * (wid * CSEL + off + j * G) + G, 256), G)])
        return 0

    lax.fori_loop(0, nb_pad // G, fl2, 0)

    cntv[pl.ds(0, 16)] = jnp.full((16,), off + nb, jnp.int32)
    pltpu.sync_copy(cntv, ocnt.at[pl.ds(pl.multiple_of(wid * 16, 16), 16)])


_preprocess = functools.partial(
    pl.kernel,
    out_type=(
        jax.ShapeDtypeStruct((2 * NW * CSEL,), jnp.int32),
        jax.ShapeDtypeStruct((NW * 16,), jnp.int32),
    ),
    mesh=_mesh,
    compiler_params=_params,
    scratch_types=[
        pltpu.VMEM((2 * KCH,), jnp.int32),
        pltpu.VMEM((2 * KCH,), jnp.int32),
        pltpu.VMEM((CAPB,), jnp.int32),
        pltpu.VMEM((CAPB,), jnp.int32),
        pltpu.VMEM((16,), jnp.int32),
        pltpu.SemaphoreType.DMA,
        pltpu.SemaphoreType.DMA,
    ],
)(_pre_body)


def _agg_body(rw, mult, h2d, opair, ocnt, out, sd, gidx, rows,
              agg, cnt16, sem0, sem1):
    wid = _wid()
    nsub = rw // 16
    pltpu.sync_copy(ocnt.at[pl.ds(pl.multiple_of(wid * 16, 16), 16)], cnt16)
    cnt = cnt16[pl.ds(0, 16)][0]
    ngroups = (cnt + G - 1) // G
    neg16 = jnp.full((16,), NEG, jnp.float32)
    sems = (sem0, sem1)

    for p in range(mult):
        def ib(r, _):
            for j in range(nsub):
                agg[r, pl.ds(j * 16, 16)] = neg16
            return 0

        lax.fori_loop(0, R + 1, ib, 0)

        def fetch(g, slot):
            pltpu.sync_copy(opair.at[pl.ds(pl.multiple_of(2 * (wid * CSEL + g * G), 256), 2 * G)],
                            sd.at[pl.ds(slot * 2 * G, 2 * G)])
            if mult > 1:
                def cv(i, _):
                    s = pl.ds(slot * 2 * G + i * 16, 16)
                    gidx[pl.ds(slot * G + i * 16, 16)] = sd[s] * mult + p
                    return 0
                lax.fori_loop(0, G // 16, cv, 0)
                for q in range(G // 128):
                    pltpu.async_copy(h2d.at[gidx.at[pl.ds(slot * G + q * 128, 128)]],
                                     rows.at[pl.ds(slot * G + q * 128, 128)],
                                     sems[slot])
            else:
                for q in range(G // 128):
                    pltpu.async_copy(h2d.at[sd.at[pl.ds(slot * 2 * G + q * 128, 128)]],
                                     rows.at[pl.ds(slot * G + q * 128, 128)],
                                     sems[slot])

        def drain_rmw(slot):
            for q in range(G // 128):
                pltpu.make_async_copy(
                    h2d.at[sd.at[pl.ds(slot * 2 * G + q * 128, 128)]],
                    rows.at[pl.ds(slot * G + q * 128, 128)],
                    sems[slot]).wait()

            def eb(e16, _):
                dv = sd[pl.ds(pl.multiple_of(slot * 2 * G + G + e16 * 16, 16), 16)]
                for t in range(16):
                    dl = dv[t]
                    e = slot * G + e16 * 16 + t
                    for j in range(nsub):
                        s = pl.ds(j * 16, 16)
                        agg[dl, s] = jnp.maximum(agg[dl, s], rows[e, s])
                return 0

            lax.fori_loop(0, G // 16, eb, 0)

        @pl.when(ngroups > 0)
        def _():
            fetch(jnp.int32(0), 0)

        def gp(i, _):
            g1 = i * 2 + 1

            @pl.when(g1 < ngroups)
            def _():
                fetch(g1, 1)

            drain_rmw(0)

            @pl.when(g1 < ngroups)
            def _():
                @pl.when(g1 + 1 < ngroups)
                def _():
                    fetch(g1 + 1, 0)

                drain_rmw(1)

            return 0

        lax.fori_loop(0, (ngroups + 1) // 2, gp, 0)

        def wb(r, _):
            for j in range(nsub):
                s = pl.ds(j * 16, 16)
                v = agg[r, s]
                agg[r, s] = jnp.where(v == NEG, jnp.float32(0.0), v)
            return 0

        lax.fori_loop(0, R, wb, 0)
        pltpu.sync_copy(agg.at[pl.ds(0, R)], out.at[p, pl.ds(pl.multiple_of(wid * R, 64), R)])


def _make_agg(rw, mult):
    return functools.partial(
        pl.kernel,
        out_type=jax.ShapeDtypeStruct((mult, NPAD, rw), jnp.float32),
        mesh=_mesh,
        compiler_params=_params,
        scratch_types=[
            pltpu.VMEM((4 * G,), jnp.int32),
            pltpu.VMEM((2 * G,), jnp.int32),
            pltpu.VMEM((2 * G, rw), jnp.float32),
            pltpu.VMEM((R + 1, rw), jnp.float32),
            pltpu.VMEM((16,), jnp.int32),
            pltpu.SemaphoreType.DMA,
            pltpu.SemaphoreType.DMA,
        ],
    )(functools.partial(_agg_body, rw, mult))


_agg_l1 = _make_agg(128, 2)
_agg_sm = _make_agg(16, 1)


def _dense1_body(a0_ref, a1_ref, x_ref, w0_ref, w1_ref, wr_ref, b_ref, o_ref):
    acc = jnp.dot(a0_ref[...], w0_ref[...], preferred_element_type=jnp.float32)
    acc += jnp.dot(a1_ref[...], w1_ref[...], preferred_element_type=jnp.float32)
    acc += jnp.dot(x_ref[...], wr_ref[...], preferred_element_type=jnp.float32)
    o_ref[...] = jnp.maximum(acc + b_ref[...], 0.0)


def _dense_body(a_ref, h_ref, wl_ref, wr_ref, b_ref, o_ref):
    acc = jnp.dot(a_ref[...], wl_ref[...], preferred_element_type=jnp.float32)
    acc += jnp.dot(h_ref[...], wr_ref[...], preferred_element_type=jnp.float32)
    o_ref[...] = jnp.maximum(acc + b_ref[...], 0.0)


def _last_body(a_ref, h_ref, wl_ref, wr_ref, b_ref, o_ref):
    acc = jnp.dot(a_ref[...], wl_ref[...], preferred_element_type=jnp.float32)
    acc += jnp.dot(h_ref[...], wr_ref[...], preferred_element_type=jnp.float32)
    z = acc + b_ref[...]
    m = jnp.max(z, axis=1, keepdims=True)
    ez = jnp.exp(z - m)
    o_ref[...] = z - m - jnp.log(jnp.sum(ez, axis=1, keepdims=True))


def _dense1(a0, a1, xp, w0, w1, wr, b):
    return pl.pallas_call(
        _dense1_body,
        out_shape=jax.ShapeDtypeStruct((NPAD, H), jnp.float32),
    )(a0, a1, xp, w0, w1, wr, b)


def _dense(a, h, wl, wr, b):
    return pl.pallas_call(
        _dense_body,
        out_shape=jax.ShapeDtypeStruct((NPAD, H), jnp.float32),
    )(a, h, wl, wr, b)


def _last(a, h, wl, wr, b):
    return pl.pallas_call(
        _last_body,
        out_shape=jax.ShapeDtypeStruct((NPAD, H), jnp.float32),
    )(a, h, wl, wr, b)


def kernel(x, edge_index, Wl1, bl1, Wr1, Wl2, bl2, Wr2, Wl3, bl3, Wr3,
           Wl4, bl4, Wr4, Wl5, bl5, Wr5, Wl6, bl6, Wr6, Wl7, bl7, Wr7):
    src = edge_index[0]
    dst = edge_index[1]
    opair, ocnt = _preprocess(src, dst)

    xp = jnp.pad(x, ((0, NPAD - N), (0, 0)))
    x2 = xp.reshape(NPAD * 2, 128)

    aggs = _agg_l1(x2, opair, ocnt)
    h = _dense1(aggs[0], aggs[1], xp,
                Wl1.T[:128, :], Wl1.T[128:, :], Wr1.T, bl1.reshape(1, H))

    layers = [(Wl2, bl2, Wr2), (Wl3, bl3, Wr3), (Wl4, bl4, Wr4),
              (Wl5, bl5, Wr5), (Wl6, bl6, Wr6)]
    for Wl, bl, Wr in layers:
        agg = _agg_sm(h, opair, ocnt)[0]
        h = _dense(agg, h, Wl.T, Wr.T, bl.reshape(1, H))

    agg = _agg_sm(h, opair, ocnt)[0]
    out = _last(agg, h, Wl7.T, Wr7.T, bl7.reshape(1, H))
    return out[:N]
